# baseline (speedup 1.0000x reference)
import functools

import jax
import jax.numpy as jnp
from jax import lax
from jax.experimental import pallas as pl
from jax.experimental.pallas import tpu as pltpu
from jax.experimental.pallas import tpu_sc as plsc

L = 17
Q = 16
NROWS = 255
NCH = 6
NW = 32
NTASK = 48
STRIDES = (L * L * L, L * L, L, 1)


def _task_scalars(wid, i):
    t = wid + NW * i
    ch = t >> 8
    h = jnp.minimum(t & 255, NROWS - 1)
    return ch, h


def _phase_a(row_a, row_b, idxbuf, wcoef, iota):
    @plsc.parallel_loop(0, 256, step=16, unroll=8)
    def _loop(p0):
        a = row_a[pl.ds(p0, 16)]
        c = row_b[pl.ds(p0, 16)]
        bidx = jnp.minimum(iota + (p0 + 1), 255)
        b = plsc.load_gather(row_a, [bidx])
        d = plsc.load_gather(row_b, [bidx])

        base = (((a >> 4) * L + (b >> 4)) * L + (c >> 4)) * L + (d >> 4)
        fa = (a & 15).astype(jnp.float32)
        fb = (b & 15).astype(jnp.float32)
        fc = (c & 15).astype(jnp.float32)
        fd = (d & 15).astype(jnp.float32)

        s1 = jnp.maximum(fa, fb); t1 = jnp.minimum(fa, fb)
        s2 = jnp.maximum(fc, fd); t2 = jnp.minimum(fc, fd)
        g1 = jnp.maximum(s1, s2); gx = jnp.minimum(s1, s2)
        gy = jnp.maximum(t1, t2); g4 = jnp.minimum(t1, t2)
        g2 = jnp.maximum(gx, gy); g3 = jnp.minimum(gx, gy)

        ra = ((fb > fa).astype(jnp.int32) + (fc > fa).astype(jnp.int32)
              + (fd > fa).astype(jnp.int32))
        rb = ((fa >= fb).astype(jnp.int32) + (fc > fb).astype(jnp.int32)
              + (fd > fb).astype(jnp.int32))
        rc = ((fa >= fc).astype(jnp.int32) + (fb >= fc).astype(jnp.int32)
              + (fd > fc).astype(jnp.int32))
        rd = ((fa >= fd).astype(jnp.int32) + (fb >= fd).astype(jnp.int32)
              + (fc >= fd).astype(jnp.int32))

        scale = 1.0 / Q
        coefs = ((float(Q) - g1) * scale, (g1 - g2) * scale,
                 (g2 - g3) * scale, (g3 - g4) * scale, g4 * scale)
        ranks = (ra, rb, rc, rd)
        for k in range(5):
            idx = base
            if k == 4:
                idx = base + sum(STRIDES)
            elif k > 0:
                for r, s in zip(ranks, STRIDES):
                    idx = idx + jnp.where(r < k, s, 0)
            flat = k * 256 + p0
            idxbuf[flat >> 7, pl.ds(flat & 127, 16)] = idx
            wcoef[pl.ds(flat, 16)] = coefs[k]


def _phase_c(wcoef, rows_v, outbuf, iota):
    @plsc.parallel_loop(0, 256, step=16, unroll=8)
    def _loop(p0):
        acc0 = jnp.zeros((16,), jnp.float32)
        acc1 = jnp.zeros((16,), jnp.float32)
        acc2 = jnp.zeros((16,), jnp.float32)
        acc3 = jnp.zeros((16,), jnp.float32)
        for k in range(5):
            flat = k * 256 + p0
            ck = wcoef[pl.ds(flat, 16)]
            d0 = iota + flat
            g0 = plsc.load_gather(rows_v, [d0, jnp.full((16,), 0, jnp.int32)])
            g1v = plsc.load_gather(rows_v, [d0, jnp.full((16,), 1, jnp.int32)])
            g2v = plsc.load_gather(rows_v, [d0, jnp.full((16,), 2, jnp.int32)])
            g3v = plsc.load_gather(rows_v, [d0, jnp.full((16,), 3, jnp.int32)])
            acc0 = acc0 + ck * g0
            acc1 = acc1 + ck * g1v
            acc2 = acc2 + ck * g2v
            acc3 = acc3 + ck * g3v
        pos = 2 * (p0 + iota)
        plsc.store_scatter(outbuf, [pos], acc0)
        plsc.store_scatter(outbuf, [pos + 1], acc1)
        plsc.store_scatter(outbuf, [pos + 512], acc2)
        plsc.store_scatter(outbuf, [pos + 513], acc3)


def _tec_body(table_hbm, img_hbm, out_hbm,
              row_a0, row_b0, row_a1, row_b1,
              idx0, idx1, wc0, wc1, rv0, rv1, ob0, ob1,
              isem0, isem1, gsem0, gsem1, osem0, osem1):
    nc = 2
    wid = lax.axis_index("s") * nc + lax.axis_index("c")
    iota = lax.iota(jnp.int32, 16)

    banks = ((row_a0, row_b0, idx0, wc0, rv0, ob0, isem0, gsem0, osem0),
             (row_a1, row_b1, idx1, wc1, rv1, ob1, isem1, gsem1, osem1))

    def fire_img(i, e):
        ch, h = _task_scalars(wid, i)
        row_a, row_b = banks[e][0], banks[e][1]
        isem = banks[e][6]
        pltpu.async_copy(img_hbm.at[ch, h], row_a, isem)
        pltpu.async_copy(img_hbm.at[ch, h + 1], row_b, isem)

    def wait_img(e):
        row_a, row_b, isem = banks[e][0], banks[e][1], banks[e][6]
        pltpu.make_async_copy(img_hbm.at[0, 0], row_a, isem).wait()
        pltpu.make_async_copy(img_hbm.at[0, 0], row_b, isem).wait()

    def fire_out(i, e):
        ch, h = _task_scalars(wid, i)
        ob, osem = banks[e][5], banks[e][8]
        pltpu.async_copy(ob.at[pl.ds(0, 512)], out_hbm.at[ch, 2 * h], osem)
        pltpu.async_copy(ob.at[pl.ds(512, 512)], out_hbm.at[ch, 2 * h + 1], osem)

    def wait_out(e):
        ob, osem = banks[e][5], banks[e][8]
        pltpu.make_async_copy(ob.at[pl.ds(0, 512)], out_hbm.at[0, 0], osem).wait()
        pltpu.make_async_copy(ob.at[pl.ds(512, 512)], out_hbm.at[0, 1], osem).wait()

    fire_img(0, 0)
    fire_img(1, 1)
    fire_out(0, 0)
    fire_out(1, 1)

    def body(j, carry):
        descs = []
        for e in (0, 1):
            i = 2 * j + e
            row_a, row_b, idxbuf, wcoef, rows_v = banks[e][:5]
            gsem = banks[e][7]
            wait_img(e)
            _phase_a(row_a, row_b, idxbuf, wcoef, iota)
            for jj in range(10):
                descs.append(pltpu.async_copy(
                    table_hbm.at[idxbuf.at[jj]],
                    rows_v.at[pl.ds(jj * 128, 128)], gsem))
            fire_img(jnp.minimum(i + 2, NTASK - 1), e)
        for e in (0, 1):
            i = 2 * j + e
            wcoef, rows_v, ob = banks[e][3], banks[e][4], banks[e][5]
            wait_out(e)
            for dsc in descs[e * 10:(e + 1) * 10]:
                dsc.wait()
            _phase_c(wcoef, rows_v, ob, iota)
            fire_out(i, e)
        return carry

    lax.fori_loop(0, NTASK // 2, body, 0)

    for e in (0, 1):
        wait_img(e)
        wait_out(e)


@functools.partial(jax.jit, static_argnames=())
def kernel(img, weight):
    B, C, H, W = img.shape
    table = jnp.pad(weight.reshape(L * L * L * L, 4), ((0, 0), (0, 4)))
    imgf = img.reshape(B * C, H, W)

    mesh = plsc.VectorSubcoreMesh(core_axis_name="c", subcore_axis_name="s")
    run = pl.kernel(
        _tec_body,
        out_type=jax.ShapeDtypeStruct((NCH, 510, 512), jnp.float32),
        mesh=mesh,
        scratch_types=[
            pltpu.VMEM((256,), jnp.int32),
            pltpu.VMEM((256,), jnp.int32),
            pltpu.VMEM((256,), jnp.int32),
            pltpu.VMEM((256,), jnp.int32),
            pltpu.VMEM((10, 128), jnp.int32),
            pltpu.VMEM((10, 128), jnp.int32),
            pltpu.VMEM((1280,), jnp.float32),
            pltpu.VMEM((1280,), jnp.float32),
            pltpu.VMEM((1280, 8), jnp.float32),
            pltpu.VMEM((1280, 8), jnp.float32),
            pltpu.VMEM((1024,), jnp.float32),
            pltpu.VMEM((1024,), jnp.float32),
            pltpu.SemaphoreType.DMA,
            pltpu.SemaphoreType.DMA,
            pltpu.SemaphoreType.DMA,
            pltpu.SemaphoreType.DMA,
            pltpu.SemaphoreType.DMA,
            pltpu.SemaphoreType.DMA,
        ],
        compiler_params=pltpu.CompilerParams(
            needs_layout_passes=False, use_tc_tiling_on_sc=False),
    )
    out = run(table, imgf)
    return out[:, :, :510].reshape(B, C, 510, 510)


# speedup vs baseline: 1.0076x; 1.0076x over previous
import functools

import jax
import jax.numpy as jnp
from jax import lax
from jax.experimental import pallas as pl
from jax.experimental.pallas import tpu as pltpu
from jax.experimental.pallas import tpu_sc as plsc

L = 17
Q = 16
NROWS = 255
NCH = 6
NW = 32
NTASK = 48
STRIDES = (L * L * L, L * L, L, 1)


def _task_scalars(wid, i):
    t = wid + NW * i
    ch = t >> 8
    h = jnp.minimum(t & 255, NROWS - 1)
    return ch, h


def _phase_a(row_a, row_b, idxbuf, wcoef, iota):
    @plsc.parallel_loop(0, 256, step=16, unroll=8)
    def _loop(p0):
        a = row_a[pl.ds(p0, 16)]
        c = row_b[pl.ds(p0, 16)]
        bidx = jnp.minimum(iota + (p0 + 1), 255)
        b = plsc.load_gather(row_a, [bidx])
        d = plsc.load_gather(row_b, [bidx])

        base = (((a >> 4) * L + (b >> 4)) * L + (c >> 4)) * L + (d >> 4)
        fa = (a & 15).astype(jnp.float32)
        fb = (b & 15).astype(jnp.float32)
        fc = (c & 15).astype(jnp.float32)
        fd = (d & 15).astype(jnp.float32)

        s1 = jnp.maximum(fa, fb); t1 = jnp.minimum(fa, fb)
        s2 = jnp.maximum(fc, fd); t2 = jnp.minimum(fc, fd)
        g1 = jnp.maximum(s1, s2); gx = jnp.minimum(s1, s2)
        gy = jnp.maximum(t1, t2); g4 = jnp.minimum(t1, t2)
        g2 = jnp.maximum(gx, gy); g3 = jnp.minimum(gx, gy)

        ra = ((fb > fa).astype(jnp.int32) + (fc > fa).astype(jnp.int32)
              + (fd > fa).astype(jnp.int32))
        rb = ((fa >= fb).astype(jnp.int32) + (fc > fb).astype(jnp.int32)
              + (fd > fb).astype(jnp.int32))
        rc = ((fa >= fc).astype(jnp.int32) + (fb >= fc).astype(jnp.int32)
              + (fd > fc).astype(jnp.int32))
        rd = ((fa >= fd).astype(jnp.int32) + (fb >= fd).astype(jnp.int32)
              + (fc >= fd).astype(jnp.int32))

        scale = 1.0 / Q
        coefs = ((float(Q) - g1) * scale, (g1 - g2) * scale,
                 (g2 - g3) * scale, (g3 - g4) * scale, g4 * scale)
        ranks = (ra, rb, rc, rd)
        for k in range(5):
            idx = base
            if k == 4:
                idx = base + sum(STRIDES)
            elif k > 0:
                for r, s in zip(ranks, STRIDES):
                    idx = idx + jnp.where(r < k, s, 0)
            flat = k * 256 + p0
            idxbuf[flat >> 7, pl.ds(flat & 127, 16)] = idx
            wcoef[pl.ds(flat, 16)] = coefs[k]


def _phase_c(wcoef, rows_v, outbuf, iota):
    @plsc.parallel_loop(0, 256, step=16, unroll=8)
    def _loop(p0):
        acc0 = jnp.zeros((16,), jnp.float32)
        acc1 = jnp.zeros((16,), jnp.float32)
        acc2 = jnp.zeros((16,), jnp.float32)
        acc3 = jnp.zeros((16,), jnp.float32)
        for k in range(5):
            flat = k * 256 + p0
            ck = wcoef[pl.ds(flat, 16)]
            d0 = iota + flat
            g0 = plsc.load_gather(rows_v, [d0, jnp.full((16,), 0, jnp.int32)])
            g1v = plsc.load_gather(rows_v, [d0, jnp.full((16,), 1, jnp.int32)])
            g2v = plsc.load_gather(rows_v, [d0, jnp.full((16,), 2, jnp.int32)])
            g3v = plsc.load_gather(rows_v, [d0, jnp.full((16,), 3, jnp.int32)])
            acc0 = acc0 + ck * g0
            acc1 = acc1 + ck * g1v
            acc2 = acc2 + ck * g2v
            acc3 = acc3 + ck * g3v
        pos = 2 * (p0 + iota)
        plsc.store_scatter(outbuf, [pos], acc0)
        plsc.store_scatter(outbuf, [pos + 1], acc1)
        plsc.store_scatter(outbuf, [pos + 512], acc2)
        plsc.store_scatter(outbuf, [pos + 513], acc3)


def _tec_body(table_hbm, img_hbm, out_hbm,
              row_a0, row_b0, row_a1, row_b1, row_a2, row_b2,
              idx0, idx1, idx2, wc0, wc1, wc2, rv0, rv1, rv2,
              ob0, ob1, ob2,
              isem0, isem1, isem2, gsem0, gsem1, gsem2,
              osem0, osem1, osem2):
    nc = 2
    wid = lax.axis_index("s") * nc + lax.axis_index("c")
    iota = lax.iota(jnp.int32, 16)

    banks = ((row_a0, row_b0, idx0, wc0, rv0, ob0, isem0, gsem0, osem0),
             (row_a1, row_b1, idx1, wc1, rv1, ob1, isem1, gsem1, osem1),
             (row_a2, row_b2, idx2, wc2, rv2, ob2, isem2, gsem2, osem2))

    def fire_img(i, e):
        ch, h = _task_scalars(wid, i)
        row_a, row_b = banks[e][0], banks[e][1]
        isem = banks[e][6]
        pltpu.async_copy(img_hbm.at[ch, h], row_a, isem)
        pltpu.async_copy(img_hbm.at[ch, h + 1], row_b, isem)

    def wait_img(e):
        row_a, row_b, isem = banks[e][0], banks[e][1], banks[e][6]
        pltpu.make_async_copy(img_hbm.at[0, 0], row_a, isem).wait()
        pltpu.make_async_copy(img_hbm.at[0, 0], row_b, isem).wait()

    def fire_out(i, e):
        ch, h = _task_scalars(wid, i)
        ob, osem = banks[e][5], banks[e][8]
        pltpu.async_copy(ob.at[pl.ds(0, 512)], out_hbm.at[ch, 2 * h], osem)
        pltpu.async_copy(ob.at[pl.ds(512, 512)], out_hbm.at[ch, 2 * h + 1], osem)

    def wait_out(e):
        ob, osem = banks[e][5], banks[e][8]
        pltpu.make_async_copy(ob.at[pl.ds(0, 512)], out_hbm.at[0, 0], osem).wait()
        pltpu.make_async_copy(ob.at[pl.ds(512, 512)], out_hbm.at[0, 1], osem).wait()

    fire_img(0, 0)
    fire_img(1, 1)
    fire_img(2, 2)
    fire_out(0, 0)
    fire_out(1, 1)
    fire_out(2, 2)

    def body(j, carry):
        descs = []
        for e in (0, 1, 2):
            i = 3 * j + e
            row_a, row_b, idxbuf, wcoef, rows_v = banks[e][:5]
            gsem = banks[e][7]
            wait_img(e)
            _phase_a(row_a, row_b, idxbuf, wcoef, iota)
            for jj in range(10):
                descs.append(pltpu.async_copy(
                    table_hbm.at[idxbuf.at[jj]],
                    rows_v.at[pl.ds(jj * 128, 128)], gsem))
            fire_img(jnp.minimum(i + 3, NTASK - 1), e)
        for e in (0, 1, 2):
            i = 3 * j + e
            wcoef, rows_v, ob = banks[e][3], banks[e][4], banks[e][5]
            wait_out(e)
            for dsc in descs[e * 10:(e + 1) * 10]:
                dsc.wait()
            _phase_c(wcoef, rows_v, ob, iota)
            fire_out(i, e)
        return carry

    lax.fori_loop(0, NTASK // 3, body, 0)

    for e in (0, 1, 2):
        wait_img(e)
        wait_out(e)


@functools.partial(jax.jit, static_argnames=())
def kernel(img, weight):
    B, C, H, W = img.shape
    table = jnp.pad(weight.reshape(L * L * L * L, 4), ((0, 0), (0, 4)))
    imgf = img.reshape(B * C, H, W)

    mesh = plsc.VectorSubcoreMesh(core_axis_name="c", subcore_axis_name="s")
    run = pl.kernel(
        _tec_body,
        out_type=jax.ShapeDtypeStruct((NCH, 510, 512), jnp.float32),
        mesh=mesh,
        scratch_types=[
            pltpu.VMEM((256,), jnp.int32),
            pltpu.VMEM((256,), jnp.int32),
            pltpu.VMEM((256,), jnp.int32),
            pltpu.VMEM((256,), jnp.int32),
            pltpu.VMEM((256,), jnp.int32),
            pltpu.VMEM((256,), jnp.int32),
            pltpu.VMEM((10, 128), jnp.int32),
            pltpu.VMEM((10, 128), jnp.int32),
            pltpu.VMEM((10, 128), jnp.int32),
            pltpu.VMEM((1280,), jnp.float32),
            pltpu.VMEM((1280,), jnp.float32),
            pltpu.VMEM((1280,), jnp.float32),
            pltpu.VMEM((1280, 8), jnp.float32),
            pltpu.VMEM((1280, 8), jnp.float32),
            pltpu.VMEM((1280, 8), jnp.float32),
            pltpu.VMEM((1024,), jnp.float32),
            pltpu.VMEM((1024,), jnp.float32),
            pltpu.VMEM((1024,), jnp.float32),
            pltpu.SemaphoreType.DMA,
            pltpu.SemaphoreType.DMA,
            pltpu.SemaphoreType.DMA,
            pltpu.SemaphoreType.DMA,
            pltpu.SemaphoreType.DMA,
            pltpu.SemaphoreType.DMA,
            pltpu.SemaphoreType.DMA,
            pltpu.SemaphoreType.DMA,
            pltpu.SemaphoreType.DMA,
        ],
        compiler_params=pltpu.CompilerParams(
            needs_layout_passes=False, use_tc_tiling_on_sc=False),
    )
    out = run(table, imgf)
    return out[:, :, :510].reshape(B, C, 510, 510)
